# Initial kernel scaffold; baseline (speedup 1.0000x reference)
import jax
import jax.numpy as jnp
from jax.experimental import pallas as pl


def kernel(x, edge_index, edge_attr, W_ne, b_ne, W_ee, b_ee, Wz, bz, Wlz, blz, Wr, br, Wlr, blr, Wh, bh, Wlh, blh, Wout, bout):
    raise NotImplementedError("write your pallas kernel here")


# speedup vs baseline: 20.9178x; 20.9178x over previous
import functools

import jax
import jax.numpy as jnp
from jax import lax
from jax.experimental import pallas as pl
from jax.experimental.pallas import tpu as pltpu
from jax.experimental.pallas import tpu_sc as plsc

_N = 10000
_E = 160000
_DN = 256
_DE = 16
_HID = 32
_NC, _NS, _L = 2, 16, 16
_NW = _NC * _NS
_EW = _E // _NW
_CH = 128
_EPW = 5120
_NCH = _EPW // _CH
_P = 10240
_RPT = _P // _NS
_NB = 400
_EB = 2000

_mesh = plsc.VectorSubcoreMesh(core_axis_name="c", subcore_axis_name="s",
                               num_cores=_NC, num_subcores=_NS)


@functools.partial(
    pl.kernel,
    out_type=jax.ShapeDtypeStruct((_NC, _P), jnp.float32),
    mesh=_mesh,
    scratch_types=[
        pltpu.VMEM_SHARED((_P,), jnp.float32),
        pltpu.VMEM((_NCH, _CH), jnp.int32),
        pltpu.VMEM((_CH,), jnp.float32),
    ],
)
def _sc_hist(dst_hbm, zeros_hbm, ones_hbm, out_hbm, hist_sh, idx_v, ones_v):
    c = lax.axis_index("c")
    s = lax.axis_index("s")
    w = c * _NS + s
    pltpu.sync_copy(zeros_hbm, hist_sh.at[pl.ds(s * _RPT, _RPT)])
    pltpu.sync_copy(ones_hbm, ones_v)
    pltpu.sync_copy(dst_hbm.at[w], idx_v)
    plsc.subcore_barrier()
    for j in range(_NCH):
        pltpu.sync_copy(ones_v, hist_sh.at[idx_v.at[j]], add=True)
    plsc.subcore_barrier()
    pltpu.sync_copy(hist_sh.at[pl.ds(s * _RPT, _RPT)],
                    out_hbm.at[c, pl.ds(s * _RPT, _RPT)])


@functools.partial(
    pl.kernel,
    out_type=jax.ShapeDtypeStruct((_NC, _P, _HID), jnp.float32),
    mesh=_mesh,
    compiler_params=pltpu.CompilerParams(use_tc_tiling_on_sc=False),
    scratch_types=[
        pltpu.VMEM_SHARED((_P, _HID), jnp.float32),
        pltpu.VMEM_SHARED((_N, _HID), jnp.float32),
        pltpu.VMEM((_NCH, _CH), jnp.int32),
        pltpu.VMEM((_NCH, _CH), jnp.int32),
        pltpu.VMEM((_CH, _HID), jnp.float32),
        pltpu.SemaphoreType.DMA,
    ],
)
def _sc_scatter(y_hbm, src_hbm, dst_hbm, zeros_hbm, out_hbm,
                acc_sh, y_sh, src_v, dst_v, rows_v, sem):
    c = lax.axis_index("c")
    s = lax.axis_index("s")
    w = c * _NS + s
    pltpu.sync_copy(zeros_hbm, acc_sh.at[pl.ds(s * _RPT, _RPT), :])

    @pl.when(s < 10)
    def _stage():
        pltpu.sync_copy(y_hbm.at[pl.ds(s * 1000, 1000), :],
                        y_sh.at[pl.ds(s * 1000, 1000), :])

    pltpu.sync_copy(src_hbm.at[w], src_v)
    pltpu.sync_copy(dst_hbm.at[w], dst_v)
    plsc.subcore_barrier()
    for j in range(_NCH):
        pltpu.async_copy(y_sh.at[src_v.at[j]], rows_v, sem).wait()
        pltpu.sync_copy(rows_v, acc_sh.at[dst_v.at[j]], add=True)
    plsc.subcore_barrier()
    pltpu.sync_copy(acc_sh.at[pl.ds(s * _RPT, _RPT), :],
                    out_hbm.at[c, pl.ds(s * _RPT, _RPT), :])


@functools.partial(
    pl.kernel,
    out_type=jax.ShapeDtypeStruct((_E,), jnp.float32),
    mesh=_mesh,
    compiler_params=pltpu.CompilerParams(needs_layout_passes=False),
    scratch_types=[
        pltpu.VMEM((_N + 16,), jnp.float32),
        pltpu.VMEM((_N + 16,), jnp.float32),
        pltpu.VMEM((_EPW,), jnp.float32),
        pltpu.VMEM((_EPW,), jnp.int32),
        pltpu.VMEM((_EPW,), jnp.int32),
        pltpu.VMEM((_EPW,), jnp.float32),
    ],
)
def _sc_edge_out(src_hbm, dst_hbm, u_hbm, v_hbm, t_hbm, out_hbm,
                 u_v, v_v, t_v, si_v, di_v, o_v):
    c = lax.axis_index("c")
    s = lax.axis_index("s")
    w = c * _NS + s
    pltpu.sync_copy(u_hbm, u_v.at[pl.ds(0, _N)])
    pltpu.sync_copy(v_hbm, v_v.at[pl.ds(0, _N)])
    pltpu.sync_copy(t_hbm.at[pl.ds(w * _EW, _EW)], t_v.at[pl.ds(0, _EW)])
    pltpu.sync_copy(src_hbm.at[pl.ds(w * _EPW, _EPW)], si_v)
    pltpu.sync_copy(dst_hbm.at[pl.ds(w * _EPW, _EPW)], di_v)

    def body(i, carry):
        off = i * _L
        gu = plsc.load_gather(u_v, [si_v[pl.ds(off, _L)]])
        gv = plsc.load_gather(v_v, [di_v[pl.ds(off, _L)]])
        o_v[pl.ds(off, _L)] = gu + gv + t_v[pl.ds(off, _L)]
        return carry

    lax.fori_loop(0, _EPW // _L, body, 0)
    pltpu.sync_copy(o_v.at[pl.ds(0, _EW)], out_hbm.at[pl.ds(w * _EW, _EW)])


def _tc_node_enc_body(x_ref, h_ref, wne_ref, bne_ref, y_ref, dinv_ref):
    h = h_ref[...]
    deg = 1.0 + h[:, 0:1] + h[:, 1:2]
    dinv = lax.rsqrt(deg)
    xe = jnp.dot(x_ref[...], wne_ref[...], preferred_element_type=jnp.float32)
    xe = jnp.maximum(xe + bne_ref[...], 0.0)
    y_ref[...] = xe * dinv
    dinv_ref[...] = dinv


_tc_node_enc = pl.pallas_call(
    _tc_node_enc_body,
    grid=(_N // _NB,),
    in_specs=[
        pl.BlockSpec((_NB, _DN), lambda i: (i, 0)),
        pl.BlockSpec((_NB, 2), lambda i: (i, 0)),
        pl.BlockSpec((_DN, _HID), lambda i: (0, 0)),
        pl.BlockSpec((1, _HID), lambda i: (0, 0)),
    ],
    out_specs=[
        pl.BlockSpec((_NB, _HID), lambda i: (i, 0)),
        pl.BlockSpec((_NB, 1), lambda i: (i, 0)),
    ],
    out_shape=[
        jax.ShapeDtypeStruct((_N, _HID), jnp.float32),
        jax.ShapeDtypeStruct((_N, 1), jnp.float32),
    ],
)


def _tc_edge_enc_body(ea_ref, wee_ref, bee_ref, wout_ref, bout_ref, t_ref):
    e = jnp.dot(ea_ref[...], wee_ref[...], preferred_element_type=jnp.float32)
    e = jnp.maximum(e + bee_ref[...], 0.0)
    w3 = wout_ref[2 * _HID:3 * _HID, :]
    t_ref[...] = jnp.dot(e, w3, preferred_element_type=jnp.float32) + bout_ref[...]


_tc_edge_enc = pl.pallas_call(
    _tc_edge_enc_body,
    grid=(_E // _EB,),
    in_specs=[
        pl.BlockSpec((_EB, _DE), lambda i: (i, 0)),
        pl.BlockSpec((_DE, _HID), lambda i: (0, 0)),
        pl.BlockSpec((1, _HID), lambda i: (0, 0)),
        pl.BlockSpec((3 * _HID, 1), lambda i: (0, 0)),
        pl.BlockSpec((1, 1), lambda i: (0, 0)),
    ],
    out_specs=pl.BlockSpec((_EB, 1), lambda i: (i, 0)),
    out_shape=jax.ShapeDtypeStruct((_E, 1), jnp.float32),
)


def _tc_node_upd_body(s0_ref, s1_ref, y_ref, dinv_ref, wz_ref, bz_ref,
                      wlz_ref, blz_ref, wh_ref, bh_ref, wlh_ref, blh_ref,
                      wout_ref, uv_ref):
    acc = (s0_ref[...] + s1_ref[...] + y_ref[...]) * dinv_ref[...]
    g1 = jnp.dot(acc, wz_ref[...], preferred_element_type=jnp.float32) + bz_ref[...]
    z = jax.nn.sigmoid(
        jnp.dot(g1, wlz_ref[0:_HID, :], preferred_element_type=jnp.float32)
        + blz_ref[...])
    g2 = jnp.dot(acc, wh_ref[...], preferred_element_type=jnp.float32) + bh_ref[...]
    ht = jnp.tanh(
        jnp.dot(g2, wlh_ref[0:_HID, :], preferred_element_type=jnp.float32)
        + blh_ref[...])
    hn = (1.0 - z) * ht
    uv_ref[...] = jnp.concatenate(
        [jnp.dot(hn, wout_ref[0:_HID, :], preferred_element_type=jnp.float32),
         jnp.dot(hn, wout_ref[_HID:2 * _HID, :], preferred_element_type=jnp.float32)],
        axis=1)


_tc_node_upd = pl.pallas_call(
    _tc_node_upd_body,
    grid=(_N // _NB,),
    in_specs=[
        pl.BlockSpec((_NB, _HID), lambda i: (i, 0)),
        pl.BlockSpec((_NB, _HID), lambda i: (i, 0)),
        pl.BlockSpec((_NB, _HID), lambda i: (i, 0)),
        pl.BlockSpec((_NB, 1), lambda i: (i, 0)),
        pl.BlockSpec((_HID, _HID), lambda i: (0, 0)),
        pl.BlockSpec((1, _HID), lambda i: (0, 0)),
        pl.BlockSpec((2 * _HID, _HID), lambda i: (0, 0)),
        pl.BlockSpec((1, _HID), lambda i: (0, 0)),
        pl.BlockSpec((_HID, _HID), lambda i: (0, 0)),
        pl.BlockSpec((1, _HID), lambda i: (0, 0)),
        pl.BlockSpec((2 * _HID, _HID), lambda i: (0, 0)),
        pl.BlockSpec((1, _HID), lambda i: (0, 0)),
        pl.BlockSpec((3 * _HID, 1), lambda i: (0, 0)),
    ],
    out_specs=pl.BlockSpec((_NB, 2), lambda i: (i, 0)),
    out_shape=jax.ShapeDtypeStruct((_N, 2), jnp.float32),
)


def kernel(x, edge_index, edge_attr, W_ne, b_ne, W_ee, b_ee, Wz, bz, Wlz, blz,
           Wr, br, Wlr, blr, Wh, bh, Wlh, blh, Wout, bout):
    f32 = jnp.float32
    src0 = edge_index[0]
    dst0 = edge_index[1]

    def pad_idx(a, fill):
        a2 = a.reshape(_NW, _EW)
        return jnp.pad(a2, ((0, 0), (0, _EPW - _EW)), constant_values=fill)

    src_pw = pad_idx(src0, 0)
    dst_pw = pad_idx(dst0, _N)
    src3 = src_pw.reshape(_NW, _NCH, _CH)
    dst3 = dst_pw.reshape(_NW, _NCH, _CH)
    src_flat = src_pw.reshape(_NW * _EPW)
    dst_flat = dst_pw.reshape(_NW * _EPW)

    zeros1 = jnp.zeros((_RPT,), f32)
    ones1 = jnp.ones((_CH,), f32)
    zeros2 = jnp.zeros((_RPT, _HID), f32)

    hist = _sc_hist(dst3, zeros1, ones1)
    hist2 = hist.T[:_N]
    y, dinv = _tc_node_enc(x, hist2, W_ne, b_ne.reshape(1, _HID))
    S = _sc_scatter(y, src3, dst3, zeros2)
    t = _tc_edge_enc(edge_attr, W_ee, b_ee.reshape(1, _HID), Wout,
                     bout.reshape(1, 1))
    uv = _tc_node_upd(S[0, :_N], S[1, :_N], y, dinv,
                      Wz, bz.reshape(1, _HID), Wlz, blz.reshape(1, _HID),
                      Wh, bh.reshape(1, _HID), Wlh, blh.reshape(1, _HID),
                      Wout)
    out = _sc_edge_out(src_flat, dst_flat, uv[:, 0], uv[:, 1],
                       t.reshape(_E))
    return out.reshape(_E, 1)


# speedup vs baseline: 21.7465x; 1.0396x over previous
import functools

import jax
import jax.numpy as jnp
from jax import lax
from jax.experimental import pallas as pl
from jax.experimental.pallas import tpu as pltpu
from jax.experimental.pallas import tpu_sc as plsc

_N = 10000
_E = 160000
_DN = 256
_DE = 16
_HID = 32
_NC, _NS, _L = 2, 16, 16
_NW = _NC * _NS
_EW = _E // _NW
_CH = 128
_EPW = 5120
_NCH = _EPW // _CH
_P = 10240
_RPT = _P // _NS
_NB = 400
_EB = 2000
_NBUF = 3

_mesh = plsc.VectorSubcoreMesh(core_axis_name="c", subcore_axis_name="s",
                               num_cores=_NC, num_subcores=_NS)
_sc_params = pltpu.CompilerParams(use_tc_tiling_on_sc=False,
                                  needs_layout_passes=False)


@functools.partial(
    pl.kernel,
    out_type=[
        jax.ShapeDtypeStruct((_NC, _P, _HID), jnp.float32),
        jax.ShapeDtypeStruct((_P,), jnp.float32),
    ],
    mesh=_mesh,
    compiler_params=_sc_params,
    scratch_types=[
        pltpu.VMEM_SHARED((_P, _HID), jnp.float32),
        pltpu.VMEM_SHARED((_P, _HID), jnp.float32),
        pltpu.VMEM_SHARED((_P,), jnp.float32),
        pltpu.VMEM((_NCH, _CH), jnp.int32),
        pltpu.VMEM((_NCH, _CH), jnp.int32),
        pltpu.VMEM((_CH,), jnp.float32),
        pltpu.VMEM((_RPT, _HID), jnp.float32),
        pltpu.VMEM((_RPT,), jnp.float32),
        pltpu.VMEM((_CH, _HID), jnp.float32),
        pltpu.VMEM((_CH, _HID), jnp.float32),
        pltpu.VMEM((_CH, _HID), jnp.float32),
        pltpu.SemaphoreType.DMA,
        pltpu.SemaphoreType.DMA,
        pltpu.SemaphoreType.DMA,
        pltpu.SemaphoreType.DMA,
        pltpu.SemaphoreType.DMA,
        pltpu.SemaphoreType.DMA,
    ],
)
def _sc_fused(xe_hbm, src_hbm, dst_hbm, zeros2_hbm, zeros1_hbm, ones_hbm,
              s_out, hist_out, acc_sh, y_sh, hist_sh, idx_a, idx_b, ones_v,
              xe_v, h_v, rows0, rows1, rows2, gs0, gs1, gs2, ss0, ss1, ss2):
    c = lax.axis_index("c")
    s = lax.axis_index("s")
    w = c * _NS + s
    rows = (rows0, rows1, rows2)
    gsem = (gs0, gs1, gs2)
    ssem = (ss0, ss1, ss2)

    pltpu.sync_copy(zeros2_hbm, acc_sh.at[pl.ds(s * _RPT, _RPT), :])
    pltpu.sync_copy(zeros1_hbm, hist_sh.at[pl.ds(s * _RPT, _RPT)])
    pltpu.sync_copy(ones_hbm, ones_v)

    @pl.when(s < 10)
    def _stage():
        pltpu.sync_copy(xe_hbm.at[pl.ds(s * 1000, 1000), :],
                        y_sh.at[pl.ds(s * 1000, 1000), :])

    plsc.subcore_barrier()

    pltpu.sync_copy(dst_hbm.at[s], idx_a)
    for j in range(_NCH):
        pltpu.sync_copy(ones_v, hist_sh.at[idx_a.at[j]], add=True)
    pltpu.sync_copy(dst_hbm.at[s + _NS], idx_a)
    for j in range(_NCH):
        pltpu.sync_copy(ones_v, hist_sh.at[idx_a.at[j]], add=True)
    plsc.subcore_barrier()

    pltpu.sync_copy(hist_sh.at[pl.ds(s * _RPT, _RPT)], h_v)
    pltpu.sync_copy(y_sh.at[pl.ds(s * _RPT, _RPT), :], xe_v)

    def dinv_body(i, carry):
        d = 1.0 + h_v[pl.ds(i * _L, _L)]
        ib = plsc.bitcast(d, jnp.int32)
        g = plsc.bitcast(jnp.int32(0x5F3759DF) - lax.shift_right_logical(ib, 1),
                         jnp.float32)
        g = g * (1.5 - 0.5 * d * g * g)
        g = g * (1.5 - 0.5 * d * g * g)
        g = g * (1.5 - 0.5 * d * g * g)
        h_v[pl.ds(i * _L, _L)] = g
        return carry

    lax.fori_loop(0, _RPT // _L, dinv_body, 0)

    def row_body(g, carry):
        dvec = h_v[pl.ds(g * _L, _L)]
        for k in range(_L):
            r = g * _L + k
            dv = dvec[k]
            xe_v[r, pl.ds(0, _L)] = xe_v[r, pl.ds(0, _L)] * dv
            xe_v[r, pl.ds(_L, _L)] = xe_v[r, pl.ds(_L, _L)] * dv
        return carry

    lax.fori_loop(0, _RPT // _L, row_body, 0)
    pltpu.sync_copy(xe_v, y_sh.at[pl.ds(s * _RPT, _RPT), :])

    @pl.when(c == 0)
    def _hist_out():
        pltpu.sync_copy(hist_sh.at[pl.ds(s * _RPT, _RPT)],
                        hist_out.at[pl.ds(s * _RPT, _RPT)])

    plsc.subcore_barrier()

    pltpu.sync_copy(src_hbm.at[w], idx_a)
    pltpu.sync_copy(dst_hbm.at[w], idx_b)
    gd = [None] * _NBUF
    sd = [None] * _NBUF
    for j in range(_NBUF):
        gd[j] = pltpu.async_copy(y_sh.at[idx_a.at[j]], rows[j], gsem[j])
    for j in range(_NCH):
        b = j % _NBUF
        gd[b].wait()
        sd[b] = pltpu.async_copy(rows[b], acc_sh.at[idx_b.at[j]], ssem[b],
                                 add=True)
        nj = j + _NBUF
        if nj < _NCH:
            sd[b].wait()
            gd[b] = pltpu.async_copy(y_sh.at[idx_a.at[nj]], rows[b], gsem[b])
            sd[b] = None
    for b in range(_NBUF):
        if sd[b] is not None:
            sd[b].wait()
    plsc.subcore_barrier()
    pltpu.sync_copy(acc_sh.at[pl.ds(s * _RPT, _RPT), :],
                    s_out.at[c, pl.ds(s * _RPT, _RPT), :])


@functools.partial(
    pl.kernel,
    out_type=jax.ShapeDtypeStruct((_E,), jnp.float32),
    mesh=_mesh,
    compiler_params=_sc_params,
    scratch_types=[
        pltpu.VMEM((_N + 16,), jnp.float32),
        pltpu.VMEM((_N + 16,), jnp.float32),
        pltpu.VMEM((_EPW,), jnp.float32),
        pltpu.VMEM((_EPW,), jnp.int32),
        pltpu.VMEM((_EPW,), jnp.int32),
        pltpu.VMEM((_EPW,), jnp.float32),
    ],
)
def _sc_edge_out(src_hbm, dst_hbm, u_hbm, v_hbm, t_hbm, out_hbm,
                 u_v, v_v, t_v, si_v, di_v, o_v):
    c = lax.axis_index("c")
    s = lax.axis_index("s")
    w = c * _NS + s
    pltpu.sync_copy(u_hbm, u_v.at[pl.ds(0, _N)])
    pltpu.sync_copy(v_hbm, v_v.at[pl.ds(0, _N)])
    pltpu.sync_copy(t_hbm.at[pl.ds(w * _EW, _EW)], t_v.at[pl.ds(0, _EW)])
    pltpu.sync_copy(src_hbm.at[pl.ds(w * _EPW, _EPW)], si_v)
    pltpu.sync_copy(dst_hbm.at[pl.ds(w * _EPW, _EPW)], di_v)

    def body(i, carry):
        off = i * _L
        gu = plsc.load_gather(u_v, [si_v[pl.ds(off, _L)]])
        gv = plsc.load_gather(v_v, [di_v[pl.ds(off, _L)]])
        o_v[pl.ds(off, _L)] = gu + gv + t_v[pl.ds(off, _L)]
        return carry

    lax.fori_loop(0, _EPW // _L, body, 0)
    pltpu.sync_copy(o_v.at[pl.ds(0, _EW)], out_hbm.at[pl.ds(w * _EW, _EW)])


def _tc_node_enc_body(x_ref, wne_ref, bne_ref, xe_ref):
    xe = jnp.dot(x_ref[...], wne_ref[...], preferred_element_type=jnp.float32)
    xe_ref[...] = jnp.maximum(xe + bne_ref[...], 0.0)


_tc_node_enc = pl.pallas_call(
    _tc_node_enc_body,
    grid=(_N // _NB,),
    in_specs=[
        pl.BlockSpec((_NB, _DN), lambda i: (i, 0)),
        pl.BlockSpec((_DN, _HID), lambda i: (0, 0)),
        pl.BlockSpec((1, _HID), lambda i: (0, 0)),
    ],
    out_specs=pl.BlockSpec((_NB, _HID), lambda i: (i, 0)),
    out_shape=jax.ShapeDtypeStruct((_N, _HID), jnp.float32),
)


def _tc_edge_enc_body(ea_ref, wee_ref, bee_ref, wout_ref, bout_ref, t_ref):
    e = jnp.dot(ea_ref[...], wee_ref[...], preferred_element_type=jnp.float32)
    e = jnp.maximum(e + bee_ref[...], 0.0)
    w3 = wout_ref[2 * _HID:3 * _HID, :]
    t_ref[...] = jnp.dot(e, w3, preferred_element_type=jnp.float32) + bout_ref[...]


_tc_edge_enc = pl.pallas_call(
    _tc_edge_enc_body,
    grid=(_E // _EB,),
    in_specs=[
        pl.BlockSpec((_EB, _DE), lambda i: (i, 0)),
        pl.BlockSpec((_DE, _HID), lambda i: (0, 0)),
        pl.BlockSpec((1, _HID), lambda i: (0, 0)),
        pl.BlockSpec((3 * _HID, 1), lambda i: (0, 0)),
        pl.BlockSpec((1, 1), lambda i: (0, 0)),
    ],
    out_specs=pl.BlockSpec((_EB, 1), lambda i: (i, 0)),
    out_shape=jax.ShapeDtypeStruct((_E, 1), jnp.float32),
)


def _tc_node_upd_body(s0_ref, s1_ref, xe_ref, h_ref, wz_ref, bz_ref,
                      wlz_ref, blz_ref, wh_ref, bh_ref, wlh_ref, blh_ref,
                      wout_ref, uv_ref):
    dinv = lax.rsqrt(1.0 + h_ref[...])
    acc = (s0_ref[...] + s1_ref[...]) * dinv + xe_ref[...] * (dinv * dinv)
    g1 = jnp.dot(acc, wz_ref[...], preferred_element_type=jnp.float32) + bz_ref[...]
    z = jax.nn.sigmoid(
        jnp.dot(g1, wlz_ref[0:_HID, :], preferred_element_type=jnp.float32)
        + blz_ref[...])
    g2 = jnp.dot(acc, wh_ref[...], preferred_element_type=jnp.float32) + bh_ref[...]
    ht = jnp.tanh(
        jnp.dot(g2, wlh_ref[0:_HID, :], preferred_element_type=jnp.float32)
        + blh_ref[...])
    hn = (1.0 - z) * ht
    uv_ref[...] = jnp.concatenate(
        [jnp.dot(hn, wout_ref[0:_HID, :], preferred_element_type=jnp.float32),
         jnp.dot(hn, wout_ref[_HID:2 * _HID, :], preferred_element_type=jnp.float32)],
        axis=1)


_tc_node_upd = pl.pallas_call(
    _tc_node_upd_body,
    grid=(_N // _NB,),
    in_specs=[
        pl.BlockSpec((_NB, _HID), lambda i: (i, 0)),
        pl.BlockSpec((_NB, _HID), lambda i: (i, 0)),
        pl.BlockSpec((_NB, _HID), lambda i: (i, 0)),
        pl.BlockSpec((_NB, 1), lambda i: (i, 0)),
        pl.BlockSpec((_HID, _HID), lambda i: (0, 0)),
        pl.BlockSpec((1, _HID), lambda i: (0, 0)),
        pl.BlockSpec((2 * _HID, _HID), lambda i: (0, 0)),
        pl.BlockSpec((1, _HID), lambda i: (0, 0)),
        pl.BlockSpec((_HID, _HID), lambda i: (0, 0)),
        pl.BlockSpec((1, _HID), lambda i: (0, 0)),
        pl.BlockSpec((2 * _HID, _HID), lambda i: (0, 0)),
        pl.BlockSpec((1, _HID), lambda i: (0, 0)),
        pl.BlockSpec((3 * _HID, 1), lambda i: (0, 0)),
    ],
    out_specs=pl.BlockSpec((_NB, 2), lambda i: (i, 0)),
    out_shape=jax.ShapeDtypeStruct((_N, 2), jnp.float32),
)


def kernel(x, edge_index, edge_attr, W_ne, b_ne, W_ee, b_ee, Wz, bz, Wlz, blz,
           Wr, br, Wlr, blr, Wh, bh, Wlh, blh, Wout, bout):
    f32 = jnp.float32
    src0 = edge_index[0]
    dst0 = edge_index[1]

    def pad_idx(a, fill):
        a2 = a.reshape(_NW, _EW)
        return jnp.pad(a2, ((0, 0), (0, _EPW - _EW)), constant_values=fill)

    src_pw = pad_idx(src0, 0)
    dst_pw = pad_idx(dst0, _N)
    src3 = src_pw.reshape(_NW, _NCH, _CH)
    dst3 = dst_pw.reshape(_NW, _NCH, _CH)
    src_flat = src_pw.reshape(_NW * _EPW)
    dst_flat = dst_pw.reshape(_NW * _EPW)

    zeros1 = jnp.zeros((_RPT,), f32)
    ones1 = jnp.ones((_CH,), f32)
    zeros2 = jnp.zeros((_RPT, _HID), f32)

    x_enc = _tc_node_enc(x, W_ne, b_ne.reshape(1, _HID))
    S, hist = _sc_fused(x_enc, src3, dst3, zeros2, zeros1, ones1)
    t = _tc_edge_enc(edge_attr, W_ee, b_ee.reshape(1, _HID), Wout,
                     bout.reshape(1, 1))
    uv = _tc_node_upd(S[0, :_N], S[1, :_N], x_enc, hist[:_N].reshape(_N, 1),
                      Wz, bz.reshape(1, _HID), Wlz, blz.reshape(1, _HID),
                      Wh, bh.reshape(1, _HID), Wlh, blh.reshape(1, _HID),
                      Wout)
    out = _sc_edge_out(src_flat, dst_flat, uv[:, 0], uv[:, 1],
                       t.reshape(_E))
    return out.reshape(_E, 1)


# speedup vs baseline: 42.0718x; 1.9346x over previous
import functools

import jax
import jax.numpy as jnp
from jax import lax
from jax.experimental import pallas as pl
from jax.experimental.pallas import tpu as pltpu
from jax.experimental.pallas import tpu_sc as plsc

_N = 10000
_E = 160000
_DN = 256
_DE = 16
_HID = 32
_NC, _NS, _L = 2, 16, 16
_NW = _NC * _NS
_EW = _E // _NW
_CH = 128
_EPW = 5120
_NCH = _EPW // _CH
_P = 10240
_RPT = _P // _NS
_NB = 1000
_EBT = 16000
_NBUF = 3

_mesh = plsc.VectorSubcoreMesh(core_axis_name="c", subcore_axis_name="s",
                               num_cores=_NC, num_subcores=_NS)
_sc_params = pltpu.CompilerParams(use_tc_tiling_on_sc=False,
                                  needs_layout_passes=False)


@functools.partial(
    pl.kernel,
    out_type=[
        jax.ShapeDtypeStruct((_NC, _P, _HID), jnp.float32),
        jax.ShapeDtypeStruct((_P,), jnp.float32),
    ],
    mesh=_mesh,
    compiler_params=_sc_params,
    scratch_types=[
        pltpu.VMEM_SHARED((_P, _HID), jnp.float32),
        pltpu.VMEM_SHARED((_P, _HID), jnp.float32),
        pltpu.VMEM_SHARED((_P,), jnp.float32),
        pltpu.VMEM((_NCH, _CH), jnp.int32),
        pltpu.VMEM((_NCH, _CH), jnp.int32),
        pltpu.VMEM((_CH,), jnp.float32),
        pltpu.VMEM((_RPT, _HID), jnp.float32),
        pltpu.VMEM((_RPT,), jnp.float32),
        pltpu.VMEM((_CH, _HID), jnp.float32),
        pltpu.VMEM((_CH, _HID), jnp.float32),
        pltpu.VMEM((_CH, _HID), jnp.float32),
        pltpu.SemaphoreType.DMA,
        pltpu.SemaphoreType.DMA,
        pltpu.SemaphoreType.DMA,
        pltpu.SemaphoreType.DMA,
        pltpu.SemaphoreType.DMA,
        pltpu.SemaphoreType.DMA,
    ],
)
def _sc_fused(xe_hbm, src_hbm, dst_hbm, zeros2_hbm, zeros1_hbm, ones_hbm,
              s_out, hist_out, acc_sh, y_sh, hist_sh, idx_a, idx_b, ones_v,
              xe_v, h_v, rows0, rows1, rows2, gs0, gs1, gs2, ss0, ss1, ss2):
    c = lax.axis_index("c")
    s = lax.axis_index("s")
    w = c * _NS + s
    rows = (rows0, rows1, rows2)
    gsem = (gs0, gs1, gs2)
    ssem = (ss0, ss1, ss2)

    pltpu.sync_copy(zeros2_hbm, acc_sh.at[pl.ds(s * _RPT, _RPT), :])
    pltpu.sync_copy(zeros1_hbm, hist_sh.at[pl.ds(s * _RPT, _RPT)])
    pltpu.sync_copy(ones_hbm, ones_v)

    @pl.when(s < 10)
    def _stage():
        pltpu.sync_copy(xe_hbm.at[pl.ds(s * 1000, 1000), :],
                        y_sh.at[pl.ds(s * 1000, 1000), :])

    plsc.subcore_barrier()

    pltpu.sync_copy(dst_hbm.at[s], idx_a)
    for j in range(_NCH):
        pltpu.sync_copy(ones_v, hist_sh.at[idx_a.at[j]], add=True)
    pltpu.sync_copy(dst_hbm.at[s + _NS], idx_a)
    for j in range(_NCH):
        pltpu.sync_copy(ones_v, hist_sh.at[idx_a.at[j]], add=True)
    plsc.subcore_barrier()

    pltpu.sync_copy(hist_sh.at[pl.ds(s * _RPT, _RPT)], h_v)
    pltpu.sync_copy(y_sh.at[pl.ds(s * _RPT, _RPT), :], xe_v)

    def dinv_body(i, carry):
        d = 1.0 + h_v[pl.ds(i * _L, _L)]
        ib = plsc.bitcast(d, jnp.int32)
        g = plsc.bitcast(jnp.int32(0x5F3759DF) - lax.shift_right_logical(ib, 1),
                         jnp.float32)
        g = g * (1.5 - 0.5 * d * g * g)
        g = g * (1.5 - 0.5 * d * g * g)
        g = g * (1.5 - 0.5 * d * g * g)
        h_v[pl.ds(i * _L, _L)] = g
        return carry

    lax.fori_loop(0, _RPT // _L, dinv_body, 0)

    def row_body(g, carry):
        dvec = h_v[pl.ds(g * _L, _L)]
        for k in range(_L):
            r = g * _L + k
            dv = dvec[k]
            xe_v[r, pl.ds(0, _L)] = xe_v[r, pl.ds(0, _L)] * dv
            xe_v[r, pl.ds(_L, _L)] = xe_v[r, pl.ds(_L, _L)] * dv
        return carry

    lax.fori_loop(0, _RPT // _L, row_body, 0)
    pltpu.sync_copy(xe_v, y_sh.at[pl.ds(s * _RPT, _RPT), :])

    @pl.when(c == 0)
    def _hist_out():
        pltpu.sync_copy(hist_sh.at[pl.ds(s * _RPT, _RPT)],
                        hist_out.at[pl.ds(s * _RPT, _RPT)])

    plsc.subcore_barrier()

    pltpu.sync_copy(src_hbm.at[w], idx_a)
    pltpu.sync_copy(dst_hbm.at[w], idx_b)
    gd = [None] * _NBUF
    sd = [None] * _NBUF
    for j in range(_NBUF):
        gd[j] = pltpu.async_copy(y_sh.at[idx_a.at[j]], rows[j], gsem[j])
    for j in range(_NCH):
        b = j % _NBUF
        gd[b].wait()
        sd[b] = pltpu.async_copy(rows[b], acc_sh.at[idx_b.at[j]], ssem[b],
                                 add=True)
        nj = j + _NBUF
        if nj < _NCH:
            sd[b].wait()
            gd[b] = pltpu.async_copy(y_sh.at[idx_a.at[nj]], rows[b], gsem[b])
            sd[b] = None
    for b in range(_NBUF):
        if sd[b] is not None:
            sd[b].wait()
    plsc.subcore_barrier()
    pltpu.sync_copy(acc_sh.at[pl.ds(s * _RPT, _RPT), :],
                    s_out.at[c, pl.ds(s * _RPT, _RPT), :])


@functools.partial(
    pl.kernel,
    out_type=jax.ShapeDtypeStruct((_E,), jnp.float32),
    mesh=_mesh,
    compiler_params=_sc_params,
    scratch_types=[
        pltpu.VMEM((_N + 16,), jnp.float32),
        pltpu.VMEM((_N + 16,), jnp.float32),
        pltpu.VMEM((_EPW,), jnp.float32),
        pltpu.VMEM((_EPW,), jnp.int32),
        pltpu.VMEM((_EPW,), jnp.int32),
        pltpu.VMEM((_EPW,), jnp.float32),
    ],
)
def _sc_edge_out(src_hbm, dst_hbm, u_hbm, v_hbm, t_hbm, out_hbm,
                 u_v, v_v, t_v, si_v, di_v, o_v):
    c = lax.axis_index("c")
    s = lax.axis_index("s")
    w = c * _NS + s
    pltpu.sync_copy(u_hbm, u_v.at[pl.ds(0, _N)])
    pltpu.sync_copy(v_hbm, v_v.at[pl.ds(0, _N)])
    pltpu.sync_copy(t_hbm.at[pl.ds(w * _EW, _EW)], t_v.at[pl.ds(0, _EW)])
    pltpu.sync_copy(src_hbm.at[pl.ds(w * _EPW, _EPW)], si_v)
    pltpu.sync_copy(dst_hbm.at[pl.ds(w * _EPW, _EPW)], di_v)

    def body(i, carry):
        off = i * _L
        gu = plsc.load_gather(u_v, [si_v[pl.ds(off, _L)]])
        gv = plsc.load_gather(v_v, [di_v[pl.ds(off, _L)]])
        o_v[pl.ds(off, _L)] = gu + gv + t_v[pl.ds(off, _L)]
        return carry

    lax.fori_loop(0, _EPW // _L, body, 0)
    pltpu.sync_copy(o_v.at[pl.ds(0, _EW)], out_hbm.at[pl.ds(w * _EW, _EW)])


def _dotT(a, bT):
    return lax.dot_general(a, bT, (((1,), (1,)), ((), ())),
                           preferred_element_type=jnp.float32)


def _tc_node_enc_body(x_ref, wneT_ref, bne_ref, xe_ref):
    xe = _dotT(x_ref[...], wneT_ref[...])
    xe_ref[...] = jnp.maximum(xe + bne_ref[...], 0.0)


_tc_node_enc = pl.pallas_call(
    _tc_node_enc_body,
    grid=(_N // _NB,),
    in_specs=[
        pl.BlockSpec((_NB, _DN), lambda i: (i, 0)),
        pl.BlockSpec((_HID, _DN), lambda i: (0, 0)),
        pl.BlockSpec((1, _HID), lambda i: (0, 0)),
    ],
    out_specs=pl.BlockSpec((_NB, _HID), lambda i: (i, 0)),
    out_shape=jax.ShapeDtypeStruct((_N, _HID), jnp.float32),
)


def _tc_edge_enc_body(eaT_ref, weeT_ref, bee_ref, w3_ref, bout_ref, t_ref):
    e = jnp.dot(weeT_ref[...], eaT_ref[...], preferred_element_type=jnp.float32)
    e = jnp.maximum(e + bee_ref[...], 0.0)
    t = jnp.dot(w3_ref[...], e, preferred_element_type=jnp.float32) + bout_ref[...]
    t_ref[pl.ds(pl.program_id(0) * _EBT, _EBT)] = t[0]


_tc_edge_enc = pl.pallas_call(
    _tc_edge_enc_body,
    grid=(_E // _EBT,),
    in_specs=[
        pl.BlockSpec((_DE, _EBT), lambda i: (0, i)),
        pl.BlockSpec((_HID, _DE), lambda i: (0, 0)),
        pl.BlockSpec((_HID, 1), lambda i: (0, 0)),
        pl.BlockSpec((1, _HID), lambda i: (0, 0)),
        pl.BlockSpec((1, 1), lambda i: (0, 0)),
    ],
    out_specs=pl.BlockSpec((_E,), lambda i: (0,)),
    out_shape=jax.ShapeDtypeStruct((_E,), jnp.float32),
)


def _tc_node_upd_body(s0_ref, s1_ref, xe_ref, h_ref, wzT_ref, bz_ref,
                      wlzT_ref, blz_ref, whT_ref, bh_ref, wlhT_ref, blh_ref,
                      woutT_ref, uv_ref):
    dinv = lax.rsqrt(1.0 + h_ref[...])
    acc = (s0_ref[...] + s1_ref[...]) * dinv + xe_ref[...] * (dinv * dinv)
    g1 = _dotT(acc, wzT_ref[...]) + bz_ref[...]
    z = jax.nn.sigmoid(_dotT(g1, wlzT_ref[:, 0:_HID]) + blz_ref[...])
    g2 = _dotT(acc, whT_ref[...]) + bh_ref[...]
    ht = jnp.tanh(_dotT(g2, wlhT_ref[:, 0:_HID]) + blh_ref[...])
    hn = (1.0 - z) * ht
    uv_ref[...] = jnp.concatenate(
        [_dotT(hn, woutT_ref[:, 0:_HID]),
         _dotT(hn, woutT_ref[:, _HID:2 * _HID])], axis=1)


_tc_node_upd = pl.pallas_call(
    _tc_node_upd_body,
    grid=(_N // _NB,),
    in_specs=[
        pl.BlockSpec((_NB, _HID), lambda i: (i, 0)),
        pl.BlockSpec((_NB, _HID), lambda i: (i, 0)),
        pl.BlockSpec((_NB, _HID), lambda i: (i, 0)),
        pl.BlockSpec((_NB, 1), lambda i: (i, 0)),
        pl.BlockSpec((_HID, _HID), lambda i: (0, 0)),
        pl.BlockSpec((1, _HID), lambda i: (0, 0)),
        pl.BlockSpec((_HID, 2 * _HID), lambda i: (0, 0)),
        pl.BlockSpec((1, _HID), lambda i: (0, 0)),
        pl.BlockSpec((_HID, _HID), lambda i: (0, 0)),
        pl.BlockSpec((1, _HID), lambda i: (0, 0)),
        pl.BlockSpec((_HID, 2 * _HID), lambda i: (0, 0)),
        pl.BlockSpec((1, _HID), lambda i: (0, 0)),
        pl.BlockSpec((1, 3 * _HID), lambda i: (0, 0)),
    ],
    out_specs=pl.BlockSpec((_NB, 2), lambda i: (i, 0)),
    out_shape=jax.ShapeDtypeStruct((_N, 2), jnp.float32),
)


def kernel(x, edge_index, edge_attr, W_ne, b_ne, W_ee, b_ee, Wz, bz, Wlz, blz,
           Wr, br, Wlr, blr, Wh, bh, Wlh, blh, Wout, bout):
    f32 = jnp.float32
    src0 = edge_index[0]
    dst0 = edge_index[1]

    def pad_idx(a, fill):
        a2 = a.reshape(_NW, _EW)
        return jnp.pad(a2, ((0, 0), (0, _EPW - _EW)), constant_values=fill)

    src_pw = pad_idx(src0, 0)
    dst_pw = pad_idx(dst0, _N)
    src3 = src_pw.reshape(_NW, _NCH, _CH)
    dst3 = dst_pw.reshape(_NW, _NCH, _CH)
    src_flat = src_pw.reshape(_NW * _EPW)
    dst_flat = dst_pw.reshape(_NW * _EPW)

    zeros1 = jnp.zeros((_RPT,), f32)
    ones1 = jnp.ones((_CH,), f32)
    zeros2 = jnp.zeros((_RPT, _HID), f32)

    x_enc = _tc_node_enc(x, W_ne.T, b_ne.reshape(1, _HID))
    S, hist = _sc_fused(x_enc, src3, dst3, zeros2, zeros1, ones1)
    t = _tc_edge_enc(edge_attr.T, W_ee.T, b_ee.reshape(_HID, 1),
                     Wout[2 * _HID:3 * _HID, 0].reshape(1, _HID),
                     bout.reshape(1, 1))
    uv = _tc_node_upd(S[0, :_N], S[1, :_N], x_enc, hist[:_N].reshape(_N, 1),
                      Wz.T, bz.reshape(1, _HID), Wlz.T, blz.reshape(1, _HID),
                      Wh.T, bh.reshape(1, _HID), Wlh.T, blh.reshape(1, _HID),
                      Wout.reshape(1, 3 * _HID))
    out = _sc_edge_out(src_flat, dst_flat, uv[:, 0], uv[:, 1], t)
    return out.reshape(_E, 1)


# speedup vs baseline: 45.0258x; 1.0702x over previous
import functools

import jax
import jax.numpy as jnp
from jax import lax
from jax.experimental import pallas as pl
from jax.experimental.pallas import tpu as pltpu
from jax.experimental.pallas import tpu_sc as plsc

_N = 10000
_E = 160000
_DN = 256
_DE = 16
_HID = 32
_NC, _NS, _L = 2, 16, 16
_NW = _NC * _NS
_EW = _E // _NW
_CH = 128
_EPW = 5120
_NCH = _EPW // _CH
_P = 10240
_RPT = _P // _NS
_NB = 1000
_EBT = 16000
_NBUF = 3

_mesh = plsc.VectorSubcoreMesh(core_axis_name="c", subcore_axis_name="s",
                               num_cores=_NC, num_subcores=_NS)
_sc_params = pltpu.CompilerParams(use_tc_tiling_on_sc=False,
                                  needs_layout_passes=False)


@functools.partial(
    pl.kernel,
    out_type=[
        jax.ShapeDtypeStruct((_NC, _P, _HID), jnp.float32),
        jax.ShapeDtypeStruct((_P,), jnp.float32),
    ],
    mesh=_mesh,
    compiler_params=_sc_params,
    scratch_types=[
        pltpu.VMEM_SHARED((_P, _HID), jnp.float32),
        pltpu.VMEM_SHARED((_P, _HID), jnp.float32),
        pltpu.VMEM_SHARED((_P,), jnp.float32),
        pltpu.VMEM((_NCH, _CH), jnp.int32),
        pltpu.VMEM((_NCH, _CH), jnp.int32),
        pltpu.VMEM((_CH,), jnp.float32),
        pltpu.VMEM((_RPT, _HID), jnp.float32),
        pltpu.VMEM((_RPT,), jnp.float32),
        pltpu.VMEM((_CH, _HID), jnp.float32),
        pltpu.VMEM((_CH, _HID), jnp.float32),
        pltpu.VMEM((_CH, _HID), jnp.float32),
        pltpu.SemaphoreType.DMA,
        pltpu.SemaphoreType.DMA,
        pltpu.SemaphoreType.DMA,
        pltpu.SemaphoreType.DMA,
        pltpu.SemaphoreType.DMA,
        pltpu.SemaphoreType.DMA,
    ],
)
def _sc_fused(xe_hbm, src_hbm, dst_hbm, zeros2_hbm, zeros1_hbm, ones_hbm,
              s_out, hist_out, acc_sh, y_sh, hist_sh, idx_a, idx_b, ones_v,
              xe_v, h_v, rows0, rows1, rows2, gs0, gs1, gs2, ss0, ss1, ss2):
    c = lax.axis_index("c")
    s = lax.axis_index("s")
    w = c * _NS + s
    rows = (rows0, rows1, rows2)
    gsem = (gs0, gs1, gs2)
    ssem = (ss0, ss1, ss2)

    pltpu.sync_copy(zeros2_hbm, acc_sh.at[pl.ds(s * _RPT, _RPT), :])
    pltpu.sync_copy(zeros1_hbm, hist_sh.at[pl.ds(s * _RPT, _RPT)])
    pltpu.sync_copy(ones_hbm, ones_v)

    @pl.when(s < 10)
    def _stage():
        pltpu.sync_copy(xe_hbm.at[pl.ds(s * 1000, 1000), :],
                        y_sh.at[pl.ds(s * 1000, 1000), :])

    plsc.subcore_barrier()

    pltpu.sync_copy(dst_hbm.at[s], idx_a)
    pltpu.sync_copy(dst_hbm.at[s + _NS], idx_b)
    hd = []
    for j in range(_NCH):
        hd.append(pltpu.async_copy(ones_v, hist_sh.at[idx_a.at[j]], gs0,
                                   add=True))
        hd.append(pltpu.async_copy(ones_v, hist_sh.at[idx_b.at[j]], gs1,
                                   add=True))
    for d in hd:
        d.wait()
    plsc.subcore_barrier()

    pltpu.sync_copy(hist_sh.at[pl.ds(s * _RPT, _RPT)], h_v)
    pltpu.sync_copy(y_sh.at[pl.ds(s * _RPT, _RPT), :], xe_v)

    def dinv_body(i, carry):
        d = 1.0 + h_v[pl.ds(i * _L, _L)]
        ib = plsc.bitcast(d, jnp.int32)
        g = plsc.bitcast(jnp.int32(0x5F3759DF) - lax.shift_right_logical(ib, 1),
                         jnp.float32)
        g = g * (1.5 - 0.5 * d * g * g)
        g = g * (1.5 - 0.5 * d * g * g)
        g = g * (1.5 - 0.5 * d * g * g)
        h_v[pl.ds(i * _L, _L)] = g
        return carry

    lax.fori_loop(0, _RPT // _L, dinv_body, 0)

    def row_body(g, carry):
        dvec = h_v[pl.ds(g * _L, _L)]
        for k in range(_L):
            r = g * _L + k
            dv = dvec[k]
            xe_v[r, pl.ds(0, _L)] = xe_v[r, pl.ds(0, _L)] * dv
            xe_v[r, pl.ds(_L, _L)] = xe_v[r, pl.ds(_L, _L)] * dv
        return carry

    lax.fori_loop(0, _RPT // _L, row_body, 0)
    pltpu.sync_copy(xe_v, y_sh.at[pl.ds(s * _RPT, _RPT), :])

    @pl.when(c == 0)
    def _hist_out():
        pltpu.sync_copy(hist_sh.at[pl.ds(s * _RPT, _RPT)],
                        hist_out.at[pl.ds(s * _RPT, _RPT)])

    plsc.subcore_barrier()

    pltpu.sync_copy(src_hbm.at[w], idx_a)
    pltpu.sync_copy(dst_hbm.at[w], idx_b)
    gd = [None] * _NBUF
    sd = [None] * _NBUF
    for j in range(_NBUF):
        gd[j] = pltpu.async_copy(y_sh.at[idx_a.at[j]], rows[j], gsem[j])
    for j in range(_NCH):
        b = j % _NBUF
        gd[b].wait()
        sd[b] = pltpu.async_copy(rows[b], acc_sh.at[idx_b.at[j]], ssem[b],
                                 add=True)
        nj = j + _NBUF
        if nj < _NCH:
            sd[b].wait()
            gd[b] = pltpu.async_copy(y_sh.at[idx_a.at[nj]], rows[b], gsem[b])
            sd[b] = None
    for b in range(_NBUF):
        if sd[b] is not None:
            sd[b].wait()
    plsc.subcore_barrier()
    pltpu.sync_copy(acc_sh.at[pl.ds(s * _RPT, _RPT), :],
                    s_out.at[c, pl.ds(s * _RPT, _RPT), :])


@functools.partial(
    pl.kernel,
    out_type=jax.ShapeDtypeStruct((_E,), jnp.float32),
    mesh=_mesh,
    compiler_params=_sc_params,
    scratch_types=[
        pltpu.VMEM((_N + 16,), jnp.float32),
        pltpu.VMEM((_N + 16,), jnp.float32),
        pltpu.VMEM((_EPW,), jnp.float32),
        pltpu.VMEM((_EPW,), jnp.int32),
        pltpu.VMEM((_EPW,), jnp.int32),
        pltpu.VMEM((_EPW,), jnp.float32),
    ],
)
def _sc_edge_out(src_hbm, dst_hbm, u_hbm, v_hbm, t_hbm, out_hbm,
                 u_v, v_v, t_v, si_v, di_v, o_v):
    c = lax.axis_index("c")
    s = lax.axis_index("s")
    w = c * _NS + s
    pltpu.sync_copy(u_hbm, u_v.at[pl.ds(0, _N)])
    pltpu.sync_copy(v_hbm, v_v.at[pl.ds(0, _N)])
    pltpu.sync_copy(t_hbm.at[pl.ds(w * _EW, _EW)], t_v.at[pl.ds(0, _EW)])
    pltpu.sync_copy(src_hbm.at[pl.ds(w * _EPW, _EPW)], si_v)
    pltpu.sync_copy(dst_hbm.at[pl.ds(w * _EPW, _EPW)], di_v)

    def body(i, carry):
        off = i * _L
        gu = plsc.load_gather(u_v, [si_v[pl.ds(off, _L)]])
        gv = plsc.load_gather(v_v, [di_v[pl.ds(off, _L)]])
        o_v[pl.ds(off, _L)] = gu + gv + t_v[pl.ds(off, _L)]
        return carry

    lax.fori_loop(0, _EPW // _L, body, 0)
    pltpu.sync_copy(o_v.at[pl.ds(0, _EW)], out_hbm.at[pl.ds(w * _EW, _EW)])


def _dotT(a, bT):
    return lax.dot_general(a, bT, (((1,), (1,)), ((), ())),
                           preferred_element_type=jnp.float32)


def _tc_node_enc_body(x_ref, wneT_ref, bne_ref, xe_ref):
    xe = _dotT(x_ref[...], wneT_ref[...])
    xe_ref[...] = jnp.maximum(xe + bne_ref[...], 0.0)


_tc_node_enc = pl.pallas_call(
    _tc_node_enc_body,
    grid=(_N // _NB,),
    in_specs=[
        pl.BlockSpec((_NB, _DN), lambda i: (i, 0)),
        pl.BlockSpec((_HID, _DN), lambda i: (0, 0)),
        pl.BlockSpec((1, _HID), lambda i: (0, 0)),
    ],
    out_specs=pl.BlockSpec((_NB, _HID), lambda i: (i, 0)),
    out_shape=jax.ShapeDtypeStruct((_N, _HID), jnp.float32),
)


def _tc_edge_enc_body(eaT_ref, weeT_ref, bee_ref, w3_ref, bout_ref, t_ref):
    e = jnp.dot(weeT_ref[...], eaT_ref[...], preferred_element_type=jnp.float32)
    e = jnp.maximum(e + bee_ref[...], 0.0)
    t = jnp.dot(w3_ref[...], e, preferred_element_type=jnp.float32) + bout_ref[...]
    t_ref[pl.ds(pl.program_id(0) * _EBT, _EBT)] = t[0]


_tc_edge_enc = pl.pallas_call(
    _tc_edge_enc_body,
    grid=(_E // _EBT,),
    in_specs=[
        pl.BlockSpec((_DE, _EBT), lambda i: (0, i)),
        pl.BlockSpec((_HID, _DE), lambda i: (0, 0)),
        pl.BlockSpec((_HID, 1), lambda i: (0, 0)),
        pl.BlockSpec((1, _HID), lambda i: (0, 0)),
        pl.BlockSpec((1, 1), lambda i: (0, 0)),
    ],
    out_specs=pl.BlockSpec((_E,), lambda i: (0,)),
    out_shape=jax.ShapeDtypeStruct((_E,), jnp.float32),
)


def _tc_node_upd_body(s_ref, xe_ref, h_ref, wzT_ref, bz_ref,
                      wlzT_ref, blz_ref, whT_ref, bh_ref, wlhT_ref, blh_ref,
                      woutT_ref, uv_ref):
    dinv = lax.rsqrt(1.0 + h_ref[...])
    acc = (s_ref[0] + s_ref[1]) * dinv + xe_ref[...] * (dinv * dinv)
    g1 = _dotT(acc, wzT_ref[...]) + bz_ref[...]
    z = jax.nn.sigmoid(_dotT(g1, wlzT_ref[:, 0:_HID]) + blz_ref[...])
    g2 = _dotT(acc, whT_ref[...]) + bh_ref[...]
    ht = jnp.tanh(_dotT(g2, wlhT_ref[:, 0:_HID]) + blh_ref[...])
    hn = (1.0 - z) * ht
    uv_ref[...] = jnp.concatenate(
        [_dotT(hn, woutT_ref[:, 0:_HID]),
         _dotT(hn, woutT_ref[:, _HID:2 * _HID])], axis=1)


_tc_node_upd = pl.pallas_call(
    _tc_node_upd_body,
    grid=(_N // _NB,),
    in_specs=[
        pl.BlockSpec((2, _NB, _HID), lambda i: (0, i, 0)),
        pl.BlockSpec((_NB, _HID), lambda i: (i, 0)),
        pl.BlockSpec((_NB, 1), lambda i: (i, 0)),
        pl.BlockSpec((_HID, _HID), lambda i: (0, 0)),
        pl.BlockSpec((1, _HID), lambda i: (0, 0)),
        pl.BlockSpec((_HID, 2 * _HID), lambda i: (0, 0)),
        pl.BlockSpec((1, _HID), lambda i: (0, 0)),
        pl.BlockSpec((_HID, _HID), lambda i: (0, 0)),
        pl.BlockSpec((1, _HID), lambda i: (0, 0)),
        pl.BlockSpec((_HID, 2 * _HID), lambda i: (0, 0)),
        pl.BlockSpec((1, _HID), lambda i: (0, 0)),
        pl.BlockSpec((1, 3 * _HID), lambda i: (0, 0)),
    ],
    out_specs=pl.BlockSpec((_NB, 2), lambda i: (i, 0)),
    out_shape=jax.ShapeDtypeStruct((_N, 2), jnp.float32),
)


def kernel(x, edge_index, edge_attr, W_ne, b_ne, W_ee, b_ee, Wz, bz, Wlz, blz,
           Wr, br, Wlr, blr, Wh, bh, Wlh, blh, Wout, bout):
    f32 = jnp.float32
    src0 = edge_index[0]
    dst0 = edge_index[1]

    def pad_idx(a, fill):
        a2 = a.reshape(_NW, _EW)
        return jnp.pad(a2, ((0, 0), (0, _EPW - _EW)), constant_values=fill)

    src_pw = pad_idx(src0, 0)
    dst_pw = pad_idx(dst0, _N)
    src3 = src_pw.reshape(_NW, _NCH, _CH)
    dst3 = dst_pw.reshape(_NW, _NCH, _CH)
    src_flat = src_pw.reshape(_NW * _EPW)
    dst_flat = dst_pw.reshape(_NW * _EPW)

    zeros1 = jnp.zeros((_RPT,), f32)
    ones1 = jnp.ones((_CH,), f32)
    zeros2 = jnp.zeros((_RPT, _HID), f32)

    x_enc = _tc_node_enc(x, W_ne.T, b_ne.reshape(1, _HID))
    S, hist = _sc_fused(x_enc, src3, dst3, zeros2, zeros1, ones1)
    t = _tc_edge_enc(edge_attr.T, W_ee.T, b_ee.reshape(_HID, 1),
                     Wout[2 * _HID:3 * _HID, 0].reshape(1, _HID),
                     bout.reshape(1, 1))
    uv = _tc_node_upd(S, x_enc, hist[:_N].reshape(_N, 1),
                      Wz.T, bz.reshape(1, _HID), Wlz.T, blz.reshape(1, _HID),
                      Wh.T, bh.reshape(1, _HID), Wlh.T, blh.reshape(1, _HID),
                      Wout.reshape(1, 3 * _HID))
    out = _sc_edge_out(src_flat, dst_flat, uv[:, 0], uv[:, 1], t)
    return out.reshape(_E, 1)
